# baseline (speedup 1.0000x reference)
import functools

import numpy as np
import jax
import jax.numpy as jnp
from jax import lax
from jax.experimental import pallas as pl
from jax.experimental.pallas import tpu as pltpu
from jax.experimental.pallas import tpu_sc as plsc

B = 131072
NUM_LEVELS = 16
LEVEL_DIM = 2
BASE_RES = 16
LOG2_HASHMAP = 19
HASH_SIZE = 1 << LOG2_HASHMAP
HP1 = int(np.uint32(2654435761).view(np.int32))
HP2 = int(np.uint32(805459861).view(np.int32))


def _level_offsets():
    offs, off = [], 0
    for i in range(NUM_LEVELS):
        res = int(np.ceil(BASE_RES * 2.0 ** i)) + 1
        n = min(HASH_SIZE, res ** 3)
        n = int(np.ceil(n / 8) * 8)
        offs.append(off)
        off += n
    offs.append(off)
    return offs


_OFF = _level_offsets()
_DENSE_LEVELS = 3
_DENSE_TOTAL = _OFF[_DENSE_LEVELS]

NC, NS = 2, 16
NW = NC * NS
CHUNK = B // NW
PBLK = 512
NBLK = CHUNK // PBLK
NV = PBLK // 16
NG = 8 * PBLK


def _grid_body(coords, emb, out, xyz_v,
               idx_a, w_a, val_a, idx_b, w_b, val_b,
               oblk_v, sem_a, sem_b):
    wid = lax.axis_index("s") * NC + lax.axis_index("c")
    bufs = ((idx_a, w_a, val_a, sem_a), (idx_b, w_b, val_b, sem_b))

    def pass_a(par, scale_f, off2, hashed, res, base):
        idx_v, w_v, _, _ = bufs[par]

        def body(j, _):
            o = j * 16
            x = xyz_v[pl.ds(o, 16)]
            y = xyz_v[pl.ds(PBLK + o, 16)]
            z = xyz_v[pl.ds(2 * PBLK + o, 16)]
            px = x * scale_f + 0.5
            py = y * scale_f + 0.5
            pz = z * scale_f + 0.5
            ix = px.astype(jnp.int32)
            iy = py.astype(jnp.int32)
            iz = pz.astype(jnp.int32)
            fx = px - ix.astype(jnp.float32)
            fy = py - iy.astype(jnp.float32)
            fz = pz - iz.astype(jnp.float32)
            wx0, wy0, wz0 = 1.0 - fx, 1.0 - fy, 1.0 - fz
            wxy = ((wx0 * wy0, fx * wy0), (wx0 * fy, fx * fy))
            wz = (wz0, fz)
            if hashed:
                hx = (ix, ix + 1)
                hy0 = iy * HP1
                hy = (hy0, hy0 + HP1)
                hz0 = iz * HP2
                hz = (hz0, hz0 + HP2)
            else:
                hx = (ix, ix + 1)
                hy0 = iy * res
                hy = (hy0, hy0 + res)
                hz0 = iz * (res * res)
                hz = (hz0, hz0 + res * res)
            for c in range(8):
                bx, by, bz = c & 1, (c >> 1) & 1, (c >> 2) & 1
                if hashed:
                    h = hx[bx] ^ hy[by] ^ hz[bz]
                    idx0 = lax.shift_left(
                        lax.bitwise_and(h, HASH_SIZE - 1), 1) + off2
                else:
                    idx0 = lax.shift_left(hx[bx] + hy[by] + hz[bz], 1) + off2
                w = wxy[by][bx] * wz[bz]
                kbase = c * PBLK + o
                idx_v[pl.ds(kbase, 16)] = idx0
                idx_v[pl.ds(NG + kbase, 16)] = idx0 + 1
                w_v[pl.ds(kbase, 16)] = w
            return 0

        lax.fori_loop(0, NV, body, 0)

    def fire(par):
        idx_v, _, val_v, sem = bufs[par]
        pltpu.async_copy(emb.at[idx_v], val_v, sem)

    def drain(par):
        idx_v, _, val_v, sem = bufs[par]
        pltpu.make_async_copy(emb.at[idx_v], val_v, sem).wait()

    def pass_b(par, l, base):
        _, w_v, val_v, _ = bufs[par]

        def body(j, _):
            o = j * 16
            acc0 = jnp.zeros((16,), jnp.float32)
            acc1 = jnp.zeros((16,), jnp.float32)
            for c in range(8):
                kbase = c * PBLK + o
                w = w_v[pl.ds(kbase, 16)]
                acc0 = acc0 + w * val_v[pl.ds(kbase, 16)]
                acc1 = acc1 + w * val_v[pl.ds(NG + kbase, 16)]
            oblk_v[pl.ds(o, 16)] = acc0
            oblk_v[pl.ds(PBLK + o, 16)] = acc1
            return 0

        lax.fori_loop(0, NV, body, 0)
        obase = (l * LEVEL_DIM) * B + base
        pltpu.sync_copy(oblk_v.at[pl.ds(0, PBLK)], out.at[pl.ds(obase, PBLK)])
        pltpu.sync_copy(oblk_v.at[pl.ds(PBLK, PBLK)],
                        out.at[pl.ds(obase + B, PBLK)])

    def hash_params(l):
        sc = lax.shift_left(BASE_RES, l)
        scale_f = sc.astype(jnp.float32) - 1.0
        off2 = (l - _DENSE_LEVELS) * (2 * HASH_SIZE) + 2 * _DENSE_TOTAL
        return scale_f, off2

    def do_block(b, _):
        base = wid * CHUNK + b * PBLK
        for d in range(3):
            pltpu.sync_copy(coords.at[pl.ds(d * B + base, PBLK)],
                            xyz_v.at[pl.ds(d * PBLK, PBLK)])

        for l in range(4):
            par = l & 1
            if l < _DENSE_LEVELS:
                res = BASE_RES << l
                pass_a(par, jnp.float32(res - 1), 2 * _OFF[l], False, res, base)
            else:
                scale_f, off2 = hash_params(jnp.int32(l))
                pass_a(par, scale_f, off2, True, 0, base)
            fire(par)
            if l > 0:
                drain(1 - par)
                pass_b(1 - par, l - 1, base)

        def pair(t, _):
            l0 = 4 + 2 * t
            for i, l in enumerate((l0, l0 + 1)):
                par = i
                scale_f, off2 = hash_params(l)
                pass_a(par, scale_f, off2, True, 0, base)
                fire(par)
                drain(1 - par)
                pass_b(1 - par, l - 1, base)
            return 0

        lax.fori_loop(0, (NUM_LEVELS - 4) // 2, pair, 0)

        drain(1)
        pass_b(1, NUM_LEVELS - 1, base)
        return 0

    lax.fori_loop(0, NBLK, do_block, 0)


_mesh = plsc.VectorSubcoreMesh(core_axis_name="c", subcore_axis_name="s")

_grid_call = functools.partial(
    pl.kernel,
    mesh=_mesh,
    out_type=jax.ShapeDtypeStruct((NUM_LEVELS * LEVEL_DIM * B,), jnp.float32),
    scratch_types=[
        pltpu.VMEM((3 * PBLK,), jnp.float32),
        pltpu.VMEM((2 * NG,), jnp.int32),
        pltpu.VMEM((NG,), jnp.float32),
        pltpu.VMEM((2 * NG,), jnp.float32),
        pltpu.VMEM((2 * NG,), jnp.int32),
        pltpu.VMEM((NG,), jnp.float32),
        pltpu.VMEM((2 * NG,), jnp.float32),
        pltpu.VMEM((LEVEL_DIM * PBLK,), jnp.float32),
        pltpu.SemaphoreType.DMA,
        pltpu.SemaphoreType.DMA,
    ],
)(_grid_body)


def kernel(inputs, embeddings):
    coords = inputs.T.reshape(-1)
    emb_flat = embeddings.reshape(-1)
    out = _grid_call(coords, emb_flat)
    out = out.reshape(NUM_LEVELS, LEVEL_DIM, B)
    return out.transpose(2, 0, 1).reshape(B, NUM_LEVELS * LEVEL_DIM)
